# baseline (speedup 1.0000x reference)
import functools

import jax
import jax.numpy as jnp
from jax import lax
from jax.experimental import pallas as pl
from jax.experimental.pallas import tpu as pltpu
from jax.experimental.pallas import tpu_sc as plsc

DIM = 64
LANES = 16
NUM_WORKERS = 32
CHUNK = 128


def _sc_embed(ids_flat, comb):
    n = ids_flat.shape[0]
    per_w = n // NUM_WORKERS
    n_chunks = per_w // CHUNK
    mesh = plsc.VectorSubcoreMesh(core_axis_name="c", subcore_axis_name="s")

    @functools.partial(
        pl.kernel,
        mesh=mesh,
        out_type=jax.ShapeDtypeStruct((n, DIM), jnp.float32),
        scratch_types=[
            pltpu.VMEM((CHUNK,), jnp.int32),
            pltpu.VMEM((CHUNK, 2 * DIM), jnp.float32),
            pltpu.VMEM((CHUNK, DIM), jnp.float32),
            pltpu.SemaphoreType.DMA,
        ],
    )
    def k(ids_hbm, comb_hbm, out_hbm, idx_v, rows_v, out_v, sem):
        wid = lax.axis_index("s") * 2 + lax.axis_index("c")
        base_off = wid * per_w

        def chunk_body(c, carry):
            off = base_off + c * CHUNK
            pltpu.sync_copy(ids_hbm.at[pl.ds(off, CHUNK)], idx_v)
            pltpu.async_copy(comb_hbm.at[idx_v], rows_v, sem).wait()

            def add_row(r, carry2):
                for j in range(DIM // LANES):
                    sl = pl.ds(j * LANES, LANES)
                    out_v[r, sl] = rows_v[r, sl] + rows_v[r, pl.ds(DIM + j * LANES, LANES)]
                return carry2

            lax.fori_loop(0, CHUNK, add_row, 0)
            pltpu.sync_copy(out_v, out_hbm.at[pl.ds(off, CHUNK)])
            return carry

        lax.fori_loop(0, n_chunks, chunk_body, 0)

    return k(ids_flat, comb)


def kernel(input_ids, base_weight, bias):
    b, l = input_ids.shape
    ids_flat = input_ids.reshape(-1).astype(jnp.int32)
    comb = jnp.concatenate([base_weight, bias], axis=1)
    out = _sc_embed(ids_flat, comb)
    return out.reshape(b, l, DIM)


# speedup vs baseline: 1.7850x; 1.7850x over previous
import functools

import jax
import jax.numpy as jnp
from jax import lax
from jax.experimental import pallas as pl
from jax.experimental.pallas import tpu as pltpu
from jax.experimental.pallas import tpu_sc as plsc

LANES = 16
N_SC = 2
N_SUB = 16
N_PASS = 2
VB = 782


def _tc_prep(base_t, bias_t):
    dim, vocab = base_t.shape
    blk = 46 * 128

    def body(b_ref, w_ref, o_ref):
        x = b_ref[...] + w_ref[...]
        for bb in range(blk // 128):
            o_ref[0, bb, :, :] = x[:, bb * 128:(bb + 1) * 128]

    return pl.pallas_call(
        body,
        grid=(dim // 8, 17),
        in_specs=[
            pl.BlockSpec((8, blk), lambda a, i: (a, i)),
            pl.BlockSpec((8, blk), lambda a, i: (a, i)),
        ],
        out_specs=pl.BlockSpec((1, blk // 128, 8, 128), lambda a, i: (a, i, 0, 0)),
        out_shape=jax.ShapeDtypeStruct((dim // 8, VB, 8, 128), jnp.float32),
    )(base_t, bias_t)


def _sc_embed_t(ids_t, w5):
    n_pos, n_batch = ids_t.shape
    n_a = w5.shape[0]
    dim = n_a * 8
    nc = n_batch // 128
    mesh = plsc.VectorSubcoreMesh(core_axis_name="c", subcore_axis_name="s")

    @functools.partial(
        pl.kernel,
        mesh=mesh,
        out_type=jax.ShapeDtypeStruct((n_pos, n_a, nc, 8 * 128), jnp.float32),
        scratch_types=[
            pltpu.VMEM((VB, 1, 128), jnp.float32),
            pltpu.VMEM((n_batch,), jnp.int32),
            pltpu.VMEM((n_batch,), jnp.int32),
            pltpu.VMEM((nc, 128), jnp.float32),
            pltpu.VMEM((nc, 128), jnp.float32),
            pltpu.SemaphoreType.DMA,
            pltpu.SemaphoreType.DMA,
            pltpu.SemaphoreType.DMA,
            pltpu.SemaphoreType.DMA,
            pltpu.SemaphoreType.DMA,
        ],
        compiler_params=pltpu.CompilerParams(
            use_tc_tiling_on_sc=False, needs_layout_passes=False),
    )
    def k(ids_hbm, w_hbm, out_hbm, lut, ids_a, ids_b, out_a, out_b,
          sem_d, sem_ia, sem_ib, sem_oa, sem_ob):
        c = lax.axis_index("c")
        s = lax.axis_index("s")
        zero16 = jnp.zeros((LANES,), jnp.int32)

        def start_ids(buf, sem, l):
            pltpu.async_copy(ids_hbm.at[l], buf, sem)

        def wait_ids(buf, sem):
            pltpu.make_async_copy(ids_hbm.at[0], buf, sem).wait()

        def out_dst(l, a, rm):
            return out_hbm.at[l, a, :, pl.ds(rm * 128, 128)]

        def gather_row(ids_buf, out_buf):
            def step(i, carry):
                for u in range(8):
                    sl = pl.ds(i * 128 + u * LANES, LANES)
                    idx = ids_buf[sl]
                    out_buf[i, pl.ds(u * LANES, LANES)] = plsc.load_gather(
                        lut,
                        [lax.shift_right_logical(idx, 7), zero16,
                         lax.bitwise_and(idx, 127)])
                return carry

            lax.fori_loop(0, nc, step, 0)

        for p in range(N_PASS):
            d = 32 * c + 16 * p + s
            a = lax.div(d, 8)
            rm = lax.rem(d, 8)

            cp = pltpu.async_copy(
                w_hbm.at[a, :, pl.ds(rm, 1)], lut, sem_d)
            start_ids(ids_a, sem_ia, 0)
            cp.wait()

            def pos_pair(j, carry):
                la = 2 * j
                start_ids(ids_b, sem_ib, la + 1)
                wait_ids(ids_a, sem_ia)

                @pl.when(j > 0)
                def _():
                    pltpu.make_async_copy(out_a, out_dst(0, a, rm),
                                          sem_oa).wait()

                gather_row(ids_a, out_a)
                pltpu.async_copy(out_a, out_dst(la, a, rm), sem_oa)

                @pl.when(j < n_pos // 2 - 1)
                def _():
                    start_ids(ids_a, sem_ia, la + 2)

                wait_ids(ids_b, sem_ib)

                @pl.when(j > 0)
                def _():
                    pltpu.make_async_copy(out_b, out_dst(0, a, rm),
                                          sem_ob).wait()

                gather_row(ids_b, out_b)
                pltpu.async_copy(out_b, out_dst(la + 1, a, rm), sem_ob)
                return carry

            lax.fori_loop(0, n_pos // 2, pos_pair, 0)
            pltpu.make_async_copy(out_a, out_dst(0, a, rm), sem_oa).wait()
            pltpu.make_async_copy(out_b, out_dst(0, a, rm), sem_ob).wait()

    return k(ids_t, w5)


def kernel(input_ids, base_weight, bias):
    n_batch, n_pos = input_ids.shape
    dim = base_weight.shape[1]
    ids_t = input_ids.astype(jnp.int32).T
    w5 = _tc_prep(base_weight.T, bias.T)
    p = _sc_embed_t(ids_t, w5)
    out = p.reshape(n_pos, 8, 32, 8, 128).transpose(2, 4, 0, 1, 3)
    return out.reshape(n_batch, n_pos, dim)


# speedup vs baseline: 2.3764x; 1.3313x over previous
import functools

import jax
import jax.numpy as jnp
from jax import lax
from jax.experimental import pallas as pl
from jax.experimental.pallas import tpu as pltpu
from jax.experimental.pallas import tpu_sc as plsc

LANES = 16
N_SC = 2
N_SUB = 16
N_PASS = 2
VB = 782


def _tc_prep(base_t, bias_t):
    dim, vocab = base_t.shape
    vb_full = vocab // 128

    def body(b_ref, w_ref, o_ref):
        x = b_ref[...] + w_ref[...]
        for bb in range(vb_full):
            o_ref[0, bb, :, :] = x[:, bb * 128:(bb + 1) * 128]
        o_ref[0, vb_full, :, 0:vocab - vb_full * 128] = x[:, vb_full * 128:]

    return pl.pallas_call(
        body,
        grid=(dim // 8,),
        in_specs=[
            pl.BlockSpec((8, vocab), lambda a: (a, 0)),
            pl.BlockSpec((8, vocab), lambda a: (a, 0)),
        ],
        out_specs=pl.BlockSpec((1, VB, 8, 128), lambda a: (a, 0, 0, 0)),
        out_shape=jax.ShapeDtypeStruct((dim // 8, VB, 8, 128), jnp.float32),
    )(base_t, bias_t)


def _sc_embed_t(ids_t, w5):
    n_pos, n_batch = ids_t.shape
    n_a = w5.shape[0]
    dim = n_a * 8
    nc = n_batch // 128
    mesh = plsc.VectorSubcoreMesh(core_axis_name="c", subcore_axis_name="s")

    @functools.partial(
        pl.kernel,
        mesh=mesh,
        out_type=jax.ShapeDtypeStruct((n_pos, n_a, nc, 8 * 128), jnp.float32),
        scratch_types=[
            pltpu.VMEM((VB, 128), jnp.float32),
            pltpu.VMEM((n_batch,), jnp.int32),
            pltpu.VMEM((n_batch,), jnp.int32),
            pltpu.VMEM((nc, 128), jnp.float32),
            pltpu.VMEM((nc, 128), jnp.float32),
            pltpu.SemaphoreType.DMA,
            pltpu.SemaphoreType.DMA,
            pltpu.SemaphoreType.DMA,
            pltpu.SemaphoreType.DMA,
            pltpu.SemaphoreType.DMA,
        ],
        compiler_params=pltpu.CompilerParams(
            use_tc_tiling_on_sc=False, needs_layout_passes=False),
    )
    def k(ids_hbm, w_hbm, out_hbm, lut, ids_a, ids_b, out_a, out_b,
          sem_d, sem_ia, sem_ib, sem_oa, sem_ob):
        c = lax.axis_index("c")
        s = lax.axis_index("s")
        zero16 = jnp.zeros((LANES,), jnp.int32)

        def start_ids(buf, sem, l):
            pltpu.async_copy(ids_hbm.at[l], buf, sem)

        def wait_ids(buf, sem):
            pltpu.make_async_copy(ids_hbm.at[0], buf, sem).wait()

        def out_dst(l, a, rm):
            return out_hbm.at[l, a, :, pl.ds(rm * 128, 128)]

        def gather_row(ids_buf, out_buf):
            def step(i, carry):
                for u in range(8):
                    sl = pl.ds(i * 128 + u * LANES, LANES)
                    idx = ids_buf[sl]
                    out_buf[i, pl.ds(u * LANES, LANES)] = plsc.load_gather(
                        lut, [zero16, idx])
                return carry

            lax.fori_loop(0, nc, step, 0)

        for p in range(N_PASS):
            d = 32 * c + 16 * p + s
            a = lax.div(d, 8)
            rm = lax.rem(d, 8)

            cp = pltpu.async_copy(
                w_hbm.at[a, :, pl.ds(rm * 128, 128)], lut, sem_d)
            start_ids(ids_a, sem_ia, 0)
            cp.wait()

            def pos_pair(j, carry):
                la = 2 * j
                start_ids(ids_b, sem_ib, la + 1)
                wait_ids(ids_a, sem_ia)

                @pl.when(j > 0)
                def _():
                    pltpu.make_async_copy(out_a, out_dst(0, a, rm),
                                          sem_oa).wait()

                gather_row(ids_a, out_a)
                pltpu.async_copy(out_a, out_dst(la, a, rm), sem_oa)

                @pl.when(j < n_pos // 2 - 1)
                def _():
                    start_ids(ids_a, sem_ia, la + 2)

                wait_ids(ids_b, sem_ib)

                @pl.when(j > 0)
                def _():
                    pltpu.make_async_copy(out_b, out_dst(0, a, rm),
                                          sem_ob).wait()

                gather_row(ids_b, out_b)
                pltpu.async_copy(out_b, out_dst(la + 1, a, rm), sem_ob)
                return carry

            lax.fori_loop(0, n_pos // 2, pos_pair, 0)
            pltpu.make_async_copy(out_a, out_dst(0, a, rm), sem_oa).wait()
            pltpu.make_async_copy(out_b, out_dst(0, a, rm), sem_ob).wait()

    return k(ids_t, w5)


def kernel(input_ids, base_weight, bias):
    n_batch, n_pos = input_ids.shape
    dim = base_weight.shape[1]
    ids_t = input_ids.astype(jnp.int32).T
    w5 = _tc_prep(base_weight.T, bias.T)
    p = _sc_embed_t(ids_t, w5.reshape(8, VB, 1024))
    out = p.reshape(n_pos, 8, 32, 8, 128).transpose(2, 4, 0, 1, 3)
    return out.reshape(n_batch, n_pos, dim)


# speedup vs baseline: 2.3856x; 1.0039x over previous
import functools

import jax
import jax.numpy as jnp
from jax import lax
from jax.experimental import pallas as pl
from jax.experimental.pallas import tpu as pltpu
from jax.experimental.pallas import tpu_sc as plsc

LANES = 16
N_SC = 2
N_SUB = 16
N_PASS = 2
VB = 782


def _tc_prep(base_t, bias_t):
    dim, vocab = base_t.shape
    vb_full = vocab // 128

    def body(b_ref, w_ref, o_ref):
        x = b_ref[...] + w_ref[...]
        for bb in range(vb_full):
            o_ref[0, bb, :, :] = x[:, bb * 128:(bb + 1) * 128]
        o_ref[0, vb_full, :, 0:vocab - vb_full * 128] = x[:, vb_full * 128:]

    return pl.pallas_call(
        body,
        grid=(dim // 8,),
        in_specs=[
            pl.BlockSpec((8, vocab), lambda a: (a, 0)),
            pl.BlockSpec((8, vocab), lambda a: (a, 0)),
        ],
        out_specs=pl.BlockSpec((1, VB, 8, 128), lambda a: (a, 0, 0, 0)),
        out_shape=jax.ShapeDtypeStruct((dim // 8, VB, 8, 128), jnp.float32),
    )(base_t, bias_t)


def _sc_embed_t(ids_t, w5):
    n_pos, n_batch = ids_t.shape
    n_a = w5.shape[0]
    dim = n_a * 8
    nc = n_batch // 128
    mesh = plsc.VectorSubcoreMesh(core_axis_name="c", subcore_axis_name="s")

    @functools.partial(
        pl.kernel,
        mesh=mesh,
        out_type=jax.ShapeDtypeStruct((n_pos, n_a, nc, 8 * 128), jnp.float32),
        scratch_types=[
            pltpu.VMEM((VB, 128), jnp.float32),
            pltpu.VMEM((n_batch,), jnp.int32),
            pltpu.VMEM((n_batch,), jnp.int32),
            pltpu.VMEM((nc, 128), jnp.float32),
            pltpu.VMEM((nc, 128), jnp.float32),
            pltpu.SemaphoreType.DMA,
            pltpu.SemaphoreType.DMA,
            pltpu.SemaphoreType.DMA,
            pltpu.SemaphoreType.DMA,
            pltpu.SemaphoreType.DMA,
        ],
        compiler_params=pltpu.CompilerParams(
            use_tc_tiling_on_sc=False, needs_layout_passes=False),
    )
    def k(ids_hbm, w_hbm, out_hbm, lut, ids_a, ids_b, out_a, out_b,
          sem_d, sem_ia, sem_ib, sem_oa, sem_ob):
        c = lax.axis_index("c")
        s = lax.axis_index("s")
        zero16 = jnp.zeros((LANES,), jnp.int32)

        def start_ids(buf, sem, l):
            pltpu.async_copy(ids_hbm.at[l], buf, sem)

        def wait_ids(buf, sem):
            pltpu.make_async_copy(ids_hbm.at[0], buf, sem).wait()

        def out_dst(l, a, rm):
            return out_hbm.at[l, a, :, pl.ds(rm * 128, 128)]

        def gather_row(ids_buf, out_buf):
            def step(i, carry):
                for v in range(4):
                    for u in range(8):
                        sl = pl.ds((i * 4 + v) * 128 + u * LANES, LANES)
                        idx = ids_buf[sl]
                        out_buf[i * 4 + v, pl.ds(u * LANES, LANES)] = (
                            plsc.load_gather(lut, [zero16, idx]))
                return carry

            lax.fori_loop(0, nc // 4, step, 0)

        for p in range(N_PASS):
            d = 32 * c + 16 * p + s
            a = lax.div(d, 8)
            rm = lax.rem(d, 8)

            cp = pltpu.async_copy(
                w_hbm.at[a, :, pl.ds(rm * 128, 128)], lut, sem_d)
            start_ids(ids_a, sem_ia, 0)
            cp.wait()

            def pos_pair(j, carry):
                la = 2 * j
                start_ids(ids_b, sem_ib, la + 1)
                wait_ids(ids_a, sem_ia)

                @pl.when(j > 0)
                def _():
                    pltpu.make_async_copy(out_a, out_dst(0, a, rm),
                                          sem_oa).wait()

                gather_row(ids_a, out_a)
                pltpu.async_copy(out_a, out_dst(la, a, rm), sem_oa)

                @pl.when(j < n_pos // 2 - 1)
                def _():
                    start_ids(ids_a, sem_ia, la + 2)

                wait_ids(ids_b, sem_ib)

                @pl.when(j > 0)
                def _():
                    pltpu.make_async_copy(out_b, out_dst(0, a, rm),
                                          sem_ob).wait()

                gather_row(ids_b, out_b)
                pltpu.async_copy(out_b, out_dst(la + 1, a, rm), sem_ob)
                return carry

            lax.fori_loop(0, n_pos // 2, pos_pair, 0)
            pltpu.make_async_copy(out_a, out_dst(0, a, rm), sem_oa).wait()
            pltpu.make_async_copy(out_b, out_dst(0, a, rm), sem_ob).wait()

    return k(ids_t, w5)


def kernel(input_ids, base_weight, bias):
    n_batch, n_pos = input_ids.shape
    dim = base_weight.shape[1]
    ids_t = input_ids.astype(jnp.int32).T
    w5 = _tc_prep(base_weight.T, bias.T)
    p = _sc_embed_t(ids_t, w5.reshape(8, VB, 1024))
    out = p.reshape(n_pos, 8, 32, 8, 128).transpose(2, 4, 0, 1, 3)
    return out.reshape(n_batch, n_pos, dim)


# speedup vs baseline: 2.5461x; 1.0673x over previous
import functools

import jax
import jax.numpy as jnp
from jax import lax
from jax.experimental import pallas as pl
from jax.experimental.pallas import tpu as pltpu
from jax.experimental.pallas import tpu_sc as plsc

LANES = 16
N_SC = 2
N_SUB = 16
N_PASS = 2
VB = 782


def _tc_prep(base_t, bias_t):
    dim, vocab = base_t.shape
    vb_full = vocab // 128

    def body(b_ref, w_ref, o_ref):
        x = b_ref[...] + w_ref[...]
        for bb in range(vb_full):
            o_ref[0, bb, :, :] = x[:, bb * 128:(bb + 1) * 128]
        o_ref[0, vb_full, :, 0:vocab - vb_full * 128] = x[:, vb_full * 128:]

    return pl.pallas_call(
        body,
        grid=(dim // 8,),
        in_specs=[
            pl.BlockSpec((8, vocab), lambda a: (a, 0)),
            pl.BlockSpec((8, vocab), lambda a: (a, 0)),
        ],
        out_specs=pl.BlockSpec((1, VB, 8, 128), lambda a: (a, 0, 0, 0)),
        out_shape=jax.ShapeDtypeStruct((dim // 8, VB, 8, 128), jnp.float32),
    )(base_t, bias_t)


def _sc_embed_t(ids_t, w5):
    n_pos, n_batch = ids_t.shape
    n_a = w5.shape[0]
    dim = n_a * 8
    nc = n_batch // 128
    mesh = plsc.VectorSubcoreMesh(core_axis_name="c", subcore_axis_name="s")

    @functools.partial(
        pl.kernel,
        mesh=mesh,
        out_type=jax.ShapeDtypeStruct((n_pos, n_a, nc, 8 * 128), jnp.float32),
        scratch_types=[
            pltpu.VMEM((VB, 128), jnp.float32),
            pltpu.VMEM((n_batch,), jnp.int32),
            pltpu.VMEM((n_batch,), jnp.int32),
            pltpu.VMEM((nc, 128), jnp.float32),
            pltpu.VMEM((nc, 128), jnp.float32),
            pltpu.VMEM_SHARED((n_pos, n_batch), jnp.int32),
            pltpu.SemaphoreType.DMA,
            pltpu.SemaphoreType.DMA,
            pltpu.SemaphoreType.DMA,
            pltpu.SemaphoreType.DMA,
            pltpu.SemaphoreType.DMA,
            pltpu.SemaphoreType.DMA,
        ],
        compiler_params=pltpu.CompilerParams(
            use_tc_tiling_on_sc=False, needs_layout_passes=False),
    )
    def k(ids_hbm, w_hbm, out_hbm, lut, ids_a, ids_b, out_a, out_b,
          ids_sh, sem_d, sem_ia, sem_ib, sem_oa, sem_ob, sem_sh):
        c = lax.axis_index("c")
        s = lax.axis_index("s")
        zero16 = jnp.zeros((LANES,), jnp.int32)

        n_stage = (n_pos + N_SUB - 1) // N_SUB
        for t in range(n_stage):
            lr = N_SUB * t + s

            @pl.when(lr < n_pos)
            def _():
                pltpu.async_copy(ids_hbm.at[lr], ids_sh.at[lr], sem_sh)

        for t in range(n_stage):
            lr = N_SUB * t + s

            @pl.when(lr < n_pos)
            def _():
                pltpu.make_async_copy(ids_hbm.at[0], ids_sh.at[0],
                                      sem_sh).wait()

        plsc.subcore_barrier()

        def start_ids(buf, sem, l):
            pltpu.async_copy(ids_sh.at[l], buf, sem)

        def wait_ids(buf, sem):
            pltpu.make_async_copy(ids_sh.at[0], buf, sem).wait()

        def out_dst(l, a, rm):
            return out_hbm.at[l, a, :, pl.ds(rm * 128, 128)]

        def gather_row(ids_buf, out_buf):
            def step(i, carry):
                for v in range(4):
                    for u in range(8):
                        sl = pl.ds((i * 4 + v) * 128 + u * LANES, LANES)
                        idx = ids_buf[sl]
                        out_buf[i * 4 + v, pl.ds(u * LANES, LANES)] = (
                            plsc.load_gather(lut, [zero16, idx]))
                return carry

            lax.fori_loop(0, nc // 4, step, 0)

        for p in range(N_PASS):
            d = 32 * c + 16 * p + s
            a = lax.div(d, 8)
            rm = lax.rem(d, 8)

            cp = pltpu.async_copy(
                w_hbm.at[a, :, pl.ds(rm * 128, 128)], lut, sem_d)
            start_ids(ids_a, sem_ia, 0)
            cp.wait()

            def pos_pair(j, carry):
                la = 2 * j
                start_ids(ids_b, sem_ib, la + 1)
                wait_ids(ids_a, sem_ia)

                @pl.when(j > 0)
                def _():
                    pltpu.make_async_copy(out_a, out_dst(0, a, rm),
                                          sem_oa).wait()

                gather_row(ids_a, out_a)
                pltpu.async_copy(out_a, out_dst(la, a, rm), sem_oa)

                @pl.when(j < n_pos // 2 - 1)
                def _():
                    start_ids(ids_a, sem_ia, la + 2)

                wait_ids(ids_b, sem_ib)

                @pl.when(j > 0)
                def _():
                    pltpu.make_async_copy(out_b, out_dst(0, a, rm),
                                          sem_ob).wait()

                gather_row(ids_b, out_b)
                pltpu.async_copy(out_b, out_dst(la + 1, a, rm), sem_ob)
                return carry

            lax.fori_loop(0, n_pos // 2, pos_pair, 0)
            pltpu.make_async_copy(out_a, out_dst(0, a, rm), sem_oa).wait()
            pltpu.make_async_copy(out_b, out_dst(0, a, rm), sem_ob).wait()

    return k(ids_t, w5)


def kernel(input_ids, base_weight, bias):
    n_batch, n_pos = input_ids.shape
    dim = base_weight.shape[1]
    ids_t = input_ids.astype(jnp.int32).T
    w5 = _tc_prep(base_weight.T, bias.T)
    p = _sc_embed_t(ids_t, w5.reshape(8, VB, 1024))
    out = p.reshape(n_pos, 8, 32, 8, 128).transpose(2, 4, 0, 1, 3)
    return out.reshape(n_batch, n_pos, dim)
